# baseline (speedup 1.0000x reference)
import functools

import jax
import jax.numpy as jnp
from jax import lax
from jax.experimental import pallas as pl
from jax.experimental.pallas import tpu as pltpu
from jax.experimental.pallas import tpu_sc as plsc

_N, _E, _D, _G = 10000, 320000, 128, 256
_H = _D // 2
_MLP_H = 128
_NPAD = 10240
_NCHUNK = 80
_CW = 128
_EPAD = 32 * _NCHUNK * _CW
_GPAD = 264
_ROWS_PER_TILE = _NPAD // 16
_ZROWS = 64
_NBUF = 5

_MESH = plsc.VectorSubcoreMesh(core_axis_name="c", subcore_axis_name="s")


@functools.partial(
    pl.kernel,
    out_type=(jax.ShapeDtypeStruct((2, _NPAD, _H), jnp.float32),
              jax.ShapeDtypeStruct((2, _NPAD, _H), jnp.float32)),
    mesh=_MESH,
    compiler_params=pltpu.CompilerParams(use_tc_tiling_on_sc=False),
    scratch_types=[
        pltpu.VMEM_SHARED((_NPAD, _H), jnp.float32),
        pltpu.VMEM((_NCHUNK, _CW), jnp.int32),
        pltpu.VMEM((_NCHUNK, _CW), jnp.int32),
        pltpu.VMEM((_NBUF, _CW, _H), jnp.float32),
        pltpu.VMEM((_ZROWS, _H), jnp.float32),
        pltpu.SemaphoreType.DMA((_NBUF,)),
        pltpu.SemaphoreType.DMA((_NBUF,)),
    ],
)
def _sc_aggregate(h0_hbm, h1_hbm, src_hbm, dst_hbm, zrows_hbm,
                  out0_hbm, out1_hbm,
                  agg_sh, src_v, dst_v, rows_v, zbuf_v, sem_g, sem_s):
    c = lax.axis_index("c")
    s = lax.axis_index("s")
    wid = c * 16 + s
    pltpu.sync_copy(src_hbm.at[wid], src_v)
    pltpu.sync_copy(dst_hbm.at[wid], dst_v)
    pltpu.sync_copy(zrows_hbm, zbuf_v)

    for h_hbm, out_hbm in ((h0_hbm, out0_hbm), (h1_hbm, out1_hbm)):
        def zstep(k, carry):
            pltpu.sync_copy(
                zbuf_v,
                agg_sh.at[pl.ds(s * _ROWS_PER_TILE + k * _ZROWS, _ZROWS)])
            return carry

        lax.fori_loop(0, _ROWS_PER_TILE // _ZROWS, zstep, 0)
        plsc.subcore_barrier()

        for b in range(_NBUF):
            pltpu.async_copy(h_hbm.at[src_v.at[b]], rows_v.at[b], sem_g.at[b])

        def outer(o, carry):
            for b in range(_NBUF):
                j = o * _NBUF + b
                pltpu.make_async_copy(h_hbm.at[src_v.at[j]], rows_v.at[b],
                                      sem_g.at[b]).wait()
                pltpu.async_copy(rows_v.at[b], agg_sh.at[dst_v.at[j]],
                                 sem_s.at[b], add=True)
                b3 = (b + 3) % _NBUF

                @pl.when((j >= 2) & (j <= _NCHUNK - 4))
                def _():
                    pltpu.make_async_copy(rows_v.at[b3],
                                          agg_sh.at[dst_v.at[j - 2]],
                                          sem_s.at[b3]).wait()
                    pltpu.async_copy(h_hbm.at[src_v.at[j + 3]], rows_v.at[b3],
                                     sem_g.at[b3])
            return carry

        lax.fori_loop(0, _NCHUNK // _NBUF, outer, 0)
        for i in range(_NBUF):
            pltpu.make_async_copy(rows_v.at[i],
                                  agg_sh.at[dst_v.at[_NCHUNK - _NBUF + i]],
                                  sem_s.at[i]).wait()
        plsc.subcore_barrier()

        def ostep(k, carry):
            base = s * _ROWS_PER_TILE + k * _ZROWS
            pltpu.sync_copy(agg_sh.at[pl.ds(base, _ZROWS)], zbuf_v)
            pltpu.sync_copy(zbuf_v, out_hbm.at[c, pl.ds(base, _ZROWS)])
            return carry

        lax.fori_loop(0, _ROWS_PER_TILE // _ZROWS, ostep, 0)
        pltpu.sync_copy(zrows_hbm, zbuf_v)


@functools.partial(
    pl.kernel,
    out_type=(jax.ShapeDtypeStruct((2, _G, _H), jnp.float32),
              jax.ShapeDtypeStruct((2, _G, _H), jnp.float32)),
    mesh=_MESH,
    compiler_params=pltpu.CompilerParams(use_tc_tiling_on_sc=False),
    scratch_types=[
        pltpu.VMEM_SHARED((_GPAD, _H), jnp.float32),
        pltpu.VMEM_SHARED((_GPAD, _H), jnp.float32),
        pltpu.VMEM((4, 80), jnp.int32),
        pltpu.VMEM((80, _H), jnp.float32),
    ],
)
def _sc_pool(h0_hbm, h1_hbm, n2g_hbm, zpool_hbm, out0_hbm, out1_hbm,
             pool0_sh, pool1_sh, n2g_v, rows_v):
    c = lax.axis_index("c")
    s = lax.axis_index("s")
    wid = c * 16 + s
    pltpu.sync_copy(n2g_hbm.at[wid], n2g_v)

    @pl.when(s == 0)
    def _():
        pltpu.sync_copy(zpool_hbm, pool0_sh)
        pltpu.sync_copy(zpool_hbm, pool1_sh)

    plsc.subcore_barrier()

    for h_hbm, pool_sh in ((h0_hbm, pool0_sh), (h1_hbm, pool1_sh)):
        def step(k, carry):
            base = wid * 320 + k * 80
            pltpu.sync_copy(h_hbm.at[pl.ds(base, 80)], rows_v)
            pltpu.sync_copy(rows_v, pool_sh.at[n2g_v.at[k]], add=True)
            return carry

        lax.fori_loop(0, 4, step, 0)

    plsc.subcore_barrier()

    @pl.when(s == 0)
    def _():
        pltpu.sync_copy(pool0_sh.at[pl.ds(0, _G)], out0_hbm.at[c])
        pltpu.sync_copy(pool1_sh.at[pl.ds(0, _G)], out1_hbm.at[c])


_BR = 1024


def _tc_layer_body(a0_ref, a1_ref, h_ref, w_ref, b_ref, r_ref, rb_ref, o_ref):
    a = jnp.concatenate([a0_ref[0] + a0_ref[1], a1_ref[0] + a1_ref[1]], axis=1)
    hf = jnp.concatenate([h_ref[0], h_ref[1]], axis=1)
    conv = jnp.dot(a, w_ref[...], preferred_element_type=jnp.float32) + b_ref[...]
    res = jnp.dot(hf, r_ref[...], preferred_element_type=jnp.float32) + rb_ref[...]
    hn = jnp.maximum(conv, 0.0) + jnp.maximum(res, 0.0)
    o_ref[0] = hn[:, :_H]
    o_ref[1] = hn[:, _H:]


_tc_layer = pl.pallas_call(
    _tc_layer_body,
    grid=(_NPAD // _BR,),
    in_specs=[
        pl.BlockSpec((2, _BR, _H), lambda i: (0, i, 0)),
        pl.BlockSpec((2, _BR, _H), lambda i: (0, i, 0)),
        pl.BlockSpec((2, _BR, _H), lambda i: (0, i, 0)),
        pl.BlockSpec((_D, _D), lambda i: (0, 0)),
        pl.BlockSpec((1, _D), lambda i: (0, 0)),
        pl.BlockSpec((_D, _D), lambda i: (0, 0)),
        pl.BlockSpec((1, _D), lambda i: (0, 0)),
    ],
    out_specs=pl.BlockSpec((2, _BR, _H), lambda i: (0, i, 0)),
    out_shape=jax.ShapeDtypeStruct((2, _NPAD, _H), jnp.float32),
)


def _tc_mlp_body(p0_ref, p1_ref, wm1_ref, bm1_ref, wm2_ref, bm2_ref, o_ref):
    p = jnp.concatenate([p0_ref[0] + p0_ref[1], p1_ref[0] + p1_ref[1]], axis=1)
    mid = jnp.maximum(
        jnp.dot(p, wm1_ref[...], preferred_element_type=jnp.float32) + bm1_ref[...],
        0.0)
    o_ref[...] = jnp.dot(mid, wm2_ref[...],
                         preferred_element_type=jnp.float32) + bm2_ref[...]


_tc_mlp = pl.pallas_call(
    _tc_mlp_body,
    out_shape=jax.ShapeDtypeStruct((_G, 1), jnp.float32),
)


def kernel(graph_feats, edge_index, node2graph,
           W1, b1, R1, rb1, W2, b2, R2, rb2, W3, b3, R3, rb3,
           Wm1, bm1, Wm2, bm2):
    f32 = jnp.float32
    hp = jnp.concatenate(
        [graph_feats, jnp.zeros((_NPAD - _N, _D), f32)], axis=0)
    h = jnp.stack([hp[:, :_H], hp[:, _H:]])
    epad = jnp.full((_EPAD - _E,), _N, jnp.int32)
    srcr = jnp.concatenate([edge_index[0], epad]).reshape(32, _NCHUNK, _CW)
    dstr = jnp.concatenate([edge_index[1], epad]).reshape(32, _NCHUNK, _CW)
    n2gr = jnp.concatenate(
        [node2graph, jnp.full((_NPAD - _N,), _G, jnp.int32)]).reshape(32, 4, 80)
    zrows = jnp.zeros((_ZROWS, _H), f32)
    zpool = jnp.zeros((_GPAD, _H), f32)

    for (W, b, R, rb) in ((W1, b1, R1, rb1), (W2, b2, R2, rb2),
                          (W3, b3, R3, rb3)):
        a0, a1 = _sc_aggregate(h[0], h[1], srcr, dstr, zrows)
        h = _tc_layer(a0, a1, h, W, b.reshape(1, _D), R, rb.reshape(1, _D))
    p0, p1 = _sc_pool(h[0], h[1], n2gr, zpool)
    return _tc_mlp(p0, p1, Wm1, bm1.reshape(1, _MLP_H), Wm2, bm2.reshape(1, 1))
